# baseline (speedup 1.0000x reference)
import functools

import jax
import jax.numpy as jnp
from jax import lax
from jax.experimental import pallas as pl
from jax.experimental.pallas import tpu as pltpu
from jax.experimental.pallas import tpu_sc as plsc

TEMP = 0.1
EPS = 1e-12

B = 4096
D = 32
DP = 128
BM = 512
CHUNK = 128
VU = 7176
VI = 10728


def _prep_body(ut_ref, it_ref, uo_ref, io_ref):
    ut = ut_ref[...]
    un = jnp.sqrt(jnp.sum(ut * ut, axis=0, keepdims=True))
    uo_ref[:, :D] = (ut / jnp.maximum(un, EPS)).T.astype(jnp.bfloat16)
    it = it_ref[...]
    inorm = jnp.sqrt(jnp.sum(it * it, axis=0, keepdims=True))
    io_ref[:, :D] = (it / jnp.maximum(inorm, EPS)).T.astype(jnp.bfloat16)


def _prep(u_table, i_table):
    return pl.pallas_call(
        _prep_body,
        out_shape=[
            jax.ShapeDtypeStruct((VU, DP), jnp.bfloat16),
            jax.ShapeDtypeStruct((VI, DP), jnp.bfloat16),
        ],
    )(u_table.T, i_table.T)


def _sc_gather(u_ids, i_ids, u_pad, i_pad):
    info = plsc.get_sparse_core_info()
    nc, ns = info.num_cores, info.num_subcores
    nw = nc * ns
    b_per_w = B // nw
    nchunk = b_per_w // CHUNK

    mesh = plsc.VectorSubcoreMesh(core_axis_name="c", subcore_axis_name="s")

    @functools.partial(
        pl.kernel,
        mesh=mesh,
        compiler_params=pltpu.CompilerParams(use_tc_tiling_on_sc=False),
        out_type=[
            jax.ShapeDtypeStruct((B, DP), jnp.bfloat16),
            jax.ShapeDtypeStruct((B, DP), jnp.bfloat16),
        ],
        scratch_types=[
            pltpu.VMEM((b_per_w,), jnp.int32),
            pltpu.VMEM((b_per_w, DP), jnp.bfloat16),
            pltpu.VMEM((b_per_w,), jnp.int32),
            pltpu.VMEM((b_per_w, DP), jnp.bfloat16),
            pltpu.SemaphoreType.DMA,
            pltpu.SemaphoreType.DMA,
        ],
    )
    def gather_k(u_ids_hbm, i_ids_hbm, u_tab_hbm, i_tab_hbm, u_out, i_out,
                 uidx_v, urows_v, iidx_v, irows_v, idsem, gsem):
        wid = lax.axis_index("s") * nc + lax.axis_index("c")
        base = wid * b_per_w
        cu = pltpu.async_copy(u_ids_hbm.at[pl.ds(base, b_per_w)], uidx_v,
                              idsem)
        ci = pltpu.async_copy(i_ids_hbm.at[pl.ds(base, b_per_w)], iidx_v,
                              idsem)
        cu.wait()
        ci.wait()
        gathers = []
        for c in range(nchunk):
            gathers.append(pltpu.async_copy(
                u_tab_hbm.at[uidx_v.at[pl.ds(c * CHUNK, CHUNK)]],
                urows_v.at[pl.ds(c * CHUNK, CHUNK)], gsem))
            gathers.append(pltpu.async_copy(
                i_tab_hbm.at[iidx_v.at[pl.ds(c * CHUNK, CHUNK)]],
                irows_v.at[pl.ds(c * CHUNK, CHUNK)], gsem))
        for g in gathers:
            g.wait()
        o0 = pltpu.async_copy(urows_v, u_out.at[pl.ds(base, b_per_w)], gsem)
        o1 = pltpu.async_copy(irows_v, i_out.at[pl.ds(base, b_per_w)], gsem)
        o0.wait()
        o1.wait()

    return gather_k(u_ids, i_ids, u_pad, i_pad)


def _tc_body(u_ref, i_ref, out_ref):
    u = u_ref[:, :D]
    i = i_ref[:, :D]
    out_ref[...] = lax.dot_general(
        u, i, (((1,), (1,)), ((), ())),
        preferred_element_type=jnp.float32,
    ) * (1.0 / TEMP)


def kernel(u_ids, i_ids, u_table, i_table):
    u_pad, i_pad = _prep(u_table, i_table)
    u_emb, i_emb = _sc_gather(
        u_ids.astype(jnp.int32), i_ids.astype(jnp.int32), u_pad, i_pad)

    return pl.pallas_call(
        _tc_body,
        grid=(B // BM,),
        in_specs=[
            pl.BlockSpec((BM, DP), lambda m: (m, 0)),
            pl.BlockSpec((B, DP), lambda m: (0, 0)),
        ],
        out_specs=pl.BlockSpec((BM, B), lambda m: (m, 0)),
        out_shape=jax.ShapeDtypeStruct((B, B), jnp.float32),
    )(u_emb, i_emb)


# speedup vs baseline: 1.3895x; 1.3895x over previous
import functools

import jax
import jax.numpy as jnp
from jax import lax
from jax.experimental import pallas as pl
from jax.experimental.pallas import tpu as pltpu
from jax.experimental.pallas import tpu_sc as plsc

TEMP = 0.1
EPS = 1e-12

B = 4096
D = 32
DP = 128
BM = 512
CHUNK = 128
VU = 7176
VI = 10728


def _prep_body(ut_ref, it_ref, uo_ref, io_ref):
    ut = ut_ref[...]
    un = jnp.sqrt(jnp.sum(ut * ut, axis=0, keepdims=True))
    uo_ref[:, :D] = (ut / jnp.maximum(un, EPS)).T
    it = it_ref[...]
    inorm = jnp.sqrt(jnp.sum(it * it, axis=0, keepdims=True))
    io_ref[:, :D] = (it / jnp.maximum(inorm, EPS)).T


def _prep(u_table, i_table):
    return pl.pallas_call(
        _prep_body,
        out_shape=[
            jax.ShapeDtypeStruct((VU, DP), jnp.float32),
            jax.ShapeDtypeStruct((VI, DP), jnp.float32),
        ],
    )(u_table.T, i_table.T)


def _sc_gather(u_ids, i_ids, u_pad, i_pad):
    info = plsc.get_sparse_core_info()
    nc, ns = info.num_cores, info.num_subcores
    nw = nc * ns
    b_per_w = B // nw
    nchunk = b_per_w // CHUNK

    mesh = plsc.VectorSubcoreMesh(core_axis_name="c", subcore_axis_name="s")

    @functools.partial(
        pl.kernel,
        mesh=mesh,
        compiler_params=pltpu.CompilerParams(use_tc_tiling_on_sc=False),
        out_type=[
            jax.ShapeDtypeStruct((B, DP), jnp.float32),
            jax.ShapeDtypeStruct((B, DP), jnp.float32),
        ],
        scratch_types=[
            pltpu.VMEM((b_per_w,), jnp.int32),
            pltpu.VMEM((b_per_w, DP), jnp.float32),
            pltpu.VMEM((b_per_w,), jnp.int32),
            pltpu.VMEM((b_per_w, DP), jnp.float32),
            pltpu.SemaphoreType.DMA,
            pltpu.SemaphoreType.DMA,
        ],
    )
    def gather_k(u_ids_hbm, i_ids_hbm, u_tab_hbm, i_tab_hbm, u_out, i_out,
                 uidx_v, urows_v, iidx_v, irows_v, idsem, gsem):
        wid = lax.axis_index("s") * nc + lax.axis_index("c")
        base = wid * b_per_w
        cu = pltpu.async_copy(u_ids_hbm.at[pl.ds(base, b_per_w)], uidx_v,
                              idsem)
        ci = pltpu.async_copy(i_ids_hbm.at[pl.ds(base, b_per_w)], iidx_v,
                              idsem)
        cu.wait()
        ci.wait()
        gathers = []
        for c in range(nchunk):
            gathers.append(pltpu.async_copy(
                u_tab_hbm.at[uidx_v.at[pl.ds(c * CHUNK, CHUNK)]],
                urows_v.at[pl.ds(c * CHUNK, CHUNK)], gsem))
            gathers.append(pltpu.async_copy(
                i_tab_hbm.at[iidx_v.at[pl.ds(c * CHUNK, CHUNK)]],
                irows_v.at[pl.ds(c * CHUNK, CHUNK)], gsem))
        for g in gathers:
            g.wait()
        o0 = pltpu.async_copy(urows_v, u_out.at[pl.ds(base, b_per_w)], gsem)
        o1 = pltpu.async_copy(irows_v, i_out.at[pl.ds(base, b_per_w)], gsem)
        o0.wait()
        o1.wait()

    return gather_k(u_ids, i_ids, u_pad, i_pad)


def _tc_body(u_ref, i_ref, out_ref):
    u = u_ref[:, :D]
    i = i_ref[:, :D]
    out_ref[...] = lax.dot_general(
        u, i, (((1,), (1,)), ((), ())),
        preferred_element_type=jnp.float32,
    ) * (1.0 / TEMP)


def kernel(u_ids, i_ids, u_table, i_table):
    u_pad, i_pad = _prep(u_table, i_table)
    u_emb, i_emb = _sc_gather(
        u_ids.astype(jnp.int32), i_ids.astype(jnp.int32), u_pad, i_pad)

    return pl.pallas_call(
        _tc_body,
        grid=(B // BM,),
        in_specs=[
            pl.BlockSpec((BM, DP), lambda m: (m, 0)),
            pl.BlockSpec((B, DP), lambda m: (0, 0)),
        ],
        out_specs=pl.BlockSpec((BM, B), lambda m: (m, 0)),
        out_shape=jax.ShapeDtypeStruct((B, B), jnp.float32),
    )(u_emb, i_emb)
